# baseline (speedup 1.0000x reference)
import functools

import jax
import jax.numpy as jnp
from jax import lax
from jax.experimental import pallas as pl
from jax.experimental.pallas import tpu as pltpu
from jax.experimental.pallas import tpu_sc as plsc

_NBUF = 5
_LANES = 16
_PACK = 8


def _sc_bag_sum(idx2, proj, num_cores, num_subcores):
    NW, TW = idx2.shape
    V, D = proj.shape
    mesh = plsc.VectorSubcoreMesh(core_axis_name="c", subcore_axis_name="s")

    def build(L):
        bpw = TW // L
        B = NW * bpw
        assert L % _NBUF == 0
        rounds = L // _NBUF
        groups = bpw // _LANES

        @functools.partial(
            pl.kernel,
            mesh=mesh,
            out_type=jax.ShapeDtypeStruct((_NBUF, B, D), jnp.float32),
            scratch_types=[
                pltpu.VMEM((TW,), jnp.int32),
                pltpu.VMEM((L, bpw), jnp.int32),
                pltpu.VMEM((_NBUF, bpw, D), jnp.float32),
            ]
            + [pltpu.SemaphoreType.DMA] * _NBUF,
            compiler_params=pltpu.CompilerParams(
                use_tc_tiling_on_sc=False, needs_layout_passes=False
            ),
        )
        def sc_bag(idx_hbm, proj_hbm, sums_hbm, idx_v, idxT_v, acc_v, *sems):
            w = lax.axis_index("s") * num_cores + lax.axis_index("c")
            pltpu.sync_copy(idx_hbm.at[w], idx_v)

            lane_base = lax.iota(jnp.int32, _LANES) * L

            def tr_body(j, carry):
                for g in range(groups):
                    vals = plsc.load_gather(idx_v, [lane_base + (g * _LANES * L + j)])
                    idxT_v[j, pl.ds(g * _LANES, _LANES)] = vals
                return carry

            lax.fori_loop(0, L, tr_body, 0)

            for k in range(_NBUF):
                pltpu.async_copy(proj_hbm.at[idxT_v.at[k]], acc_v.at[k], sems[k])

            def round_body(r, carry):
                for k in range(_NBUF):
                    j = r * _NBUF + k
                    pltpu.make_async_copy(
                        proj_hbm.at[idxT_v.at[k]], acc_v.at[k], sems[k]
                    ).wait()
                    pltpu.async_copy(
                        proj_hbm.at[idxT_v.at[j]], acc_v.at[k], sems[k], add=True
                    )
                return carry

            lax.fori_loop(1, rounds, round_body, 0)

            base = w * bpw
            for k in range(_NBUF):
                pltpu.make_async_copy(
                    proj_hbm.at[idxT_v.at[k]], acc_v.at[k], sems[k]
                ).wait()
                pltpu.sync_copy(acc_v.at[k], sums_hbm.at[k].at[pl.ds(base, bpw)])

        return sc_bag

    return build


def kernel(text, offsets, table, W_fc, b_fc):
    T = text.shape[0]
    B = offsets.shape[0]
    L = T // B
    V, D = table.shape
    NCAT = W_fc.shape[0]
    tableT = table.T
    inv_l = 1.0 / float(L)

    VB = 65536
    NB = -(-V // VB)
    Vpad = NB * VB

    def proj_body(tt_ref, w_ref, out_ref):
        w_scaled = w_ref[...] * inv_l
        out_ref[...] = lax.dot_general(
            w_scaled,
            tt_ref[...],
            (((1,), (0,)), ((), ())),
            preferred_element_type=jnp.float32,
        )

    projT = pl.pallas_call(
        proj_body,
        grid=(NB,),
        in_specs=[
            pl.BlockSpec((D, VB), lambda i: (0, i)),
            pl.BlockSpec((NCAT, D), lambda i: (0, 0)),
        ],
        out_specs=pl.BlockSpec((NCAT, VB), lambda i: (0, i)),
        out_shape=jax.ShapeDtypeStruct((NCAT, Vpad), jnp.float32),
    )(tableT, W_fc)
    projV = projT.T

    info = plsc.get_sparse_core_info()
    NW = info.num_cores * info.num_subcores
    bpw = B // NW

    idx2 = text.reshape(NW, bpw * L)
    sc_bag = _sc_bag_sum(idx2, projV, info.num_cores, info.num_subcores)(L)
    sums = sc_bag(idx2, projV)

    def merge_body(s_ref, b_ref, out_ref):
        out_ref[...] = jnp.sum(s_ref[...], axis=0) + b_ref[...]

    out = pl.pallas_call(
        merge_body,
        out_shape=jax.ShapeDtypeStruct((B, NCAT), jnp.float32),
    )(sums, b_fc.reshape(1, NCAT))
    return out


# speedup vs baseline: 1.2159x; 1.2159x over previous
import functools

import jax
import jax.numpy as jnp
from jax import lax
from jax.experimental import pallas as pl
from jax.experimental.pallas import tpu as pltpu
from jax.experimental.pallas import tpu_sc as plsc

_NBUF = 5
_LANES = 16
_PACK = 8


def _sc_bag_sum(idx2, proj, num_cores, num_subcores):
    NW, TW = idx2.shape
    V, D = proj.shape
    mesh = plsc.VectorSubcoreMesh(core_axis_name="c", subcore_axis_name="s")

    def build(L):
        bpw = TW // L
        B = NW * bpw
        assert L % _NBUF == 0
        rounds = L // _NBUF
        groups = bpw // _LANES

        @functools.partial(
            pl.kernel,
            mesh=mesh,
            out_type=jax.ShapeDtypeStruct((_NBUF, B, D), jnp.float32),
            scratch_types=[
                pltpu.VMEM((TW,), jnp.int32),
                pltpu.VMEM((L, bpw), jnp.int32),
                pltpu.VMEM((_NBUF, bpw, D), jnp.float32),
            ]
            + [pltpu.SemaphoreType.DMA] * _NBUF,
            compiler_params=pltpu.CompilerParams(
                use_tc_tiling_on_sc=False, needs_layout_passes=False
            ),
        )
        def sc_bag(idx_hbm, proj_hbm, sums_hbm, idx_v, idxT_v, acc_v, *sems):
            w = lax.axis_index("s") * num_cores + lax.axis_index("c")
            pltpu.sync_copy(idx_hbm.at[w], idx_v)

            lane_base = lax.iota(jnp.int32, _LANES) * L

            def tr_body(j, carry):
                for g in range(groups):
                    vals = plsc.load_gather(idx_v, [lane_base + (g * _LANES * L + j)])
                    idxT_v[j, pl.ds(g * _LANES, _LANES)] = vals
                return carry

            lax.fori_loop(0, L, tr_body, 0)

            for k in range(_NBUF):
                pltpu.async_copy(proj_hbm.at[idxT_v.at[k]], acc_v.at[k], sems[k])

            def round_body(r, carry):
                for k in range(_NBUF):
                    j = r * _NBUF + k
                    pltpu.make_async_copy(
                        proj_hbm.at[idxT_v.at[k]], acc_v.at[k], sems[k]
                    ).wait()
                    pltpu.async_copy(
                        proj_hbm.at[idxT_v.at[j]], acc_v.at[k], sems[k], add=True
                    )
                return carry

            lax.fori_loop(1, rounds, round_body, 0)

            base = w * bpw
            for k in range(_NBUF):
                pltpu.make_async_copy(
                    proj_hbm.at[idxT_v.at[k]], acc_v.at[k], sems[k]
                ).wait()
                pltpu.sync_copy(acc_v.at[k], sums_hbm.at[k].at[pl.ds(base, bpw)])

        return sc_bag

    return build


def kernel(text, offsets, table, W_fc, b_fc):
    T = text.shape[0]
    B = offsets.shape[0]
    L = T // B
    V, D = table.shape
    NCAT = W_fc.shape[0]
    tableT = table.T
    inv_l = 1.0 / float(L)

    VB = 32768
    NB = -(-V // VB)
    Vpad = NB * VB

    def proj_body(tt_ref, w_ref, out_ref):
        w_scaled = w_ref[...] * inv_l
        out_ref[...] = lax.dot_general(
            tt_ref[...],
            w_scaled,
            (((0,), (1,)), ((), ())),
            preferred_element_type=jnp.float32,
        )

    projV = pl.pallas_call(
        proj_body,
        grid=(NB,),
        in_specs=[
            pl.BlockSpec((D, VB), lambda i: (0, i)),
            pl.BlockSpec((NCAT, D), lambda i: (0, 0)),
        ],
        out_specs=pl.BlockSpec((VB, NCAT), lambda i: (i, 0)),
        out_shape=jax.ShapeDtypeStruct((Vpad, NCAT), jnp.float32),
    )(tableT, W_fc)

    info = plsc.get_sparse_core_info()
    NW = info.num_cores * info.num_subcores
    bpw = B // NW

    idx2 = text.reshape(NW, bpw * L)
    sc_bag = _sc_bag_sum(idx2, projV, info.num_cores, info.num_subcores)(L)
    sums = sc_bag(idx2, projV)

    def merge_body(s_ref, b_ref, out_ref):
        out_ref[...] = jnp.sum(s_ref[...], axis=0) + b_ref[...]

    out = pl.pallas_call(
        merge_body,
        out_shape=jax.ShapeDtypeStruct((B, NCAT), jnp.float32),
    )(sums, b_fc.reshape(1, NCAT))
    return out
